# baseline (speedup 1.0000x reference)
import functools

import jax
import jax.numpy as jnp
from jax import lax
from jax.experimental import pallas as pl
from jax.experimental.pallas import tpu as pltpu
from jax.experimental.pallas import tpu_sc as plsc

N_USERS = 50000
N_ITEMS = 50000
N_WARM = 40000
N_EDGES = 1600000
D0 = 256
D1 = 128
HID = 128
EMB = 32

R = 2000
NB = N_ITEMS // R

NC = 2
NS = 16
NW = NC * NS
SUBW = 128
NSUB = 8
C = SUBW * NSUB
NCHUNK = 100
EPT = C * NCHUNK
N_EDGES_P = EPT * NS
ROWS_PT = EPT // SUBW
HEMB = EMB // NC
NBUF = 3
WNS = 16
WCHUNK = (N_EDGES_P // SUBW) // NW // WNS
RPT = 3128
N_USERS_P = RPT * NS
WCH = 184
NWCH = RPT // WCH
EROWS = N_EDGES_P // SUBW

_EPS_N = 1e-12
_EPS_BN = 1e-5


def _l2n_rows(x):
    n = jnp.sqrt(jnp.sum(x * x, axis=-1, keepdims=True))
    return x / jnp.maximum(n, _EPS_N)


def _p1_body(f0_ref, f1_ref, W0_ref, b0_ref, W1_ref, b1_ref,
             x0_ref, x1_ref, st_ref):
    i = pl.program_id(0)
    f0 = _l2n_rows(f0_ref[...])
    x0 = jnp.dot(f0, W0_ref[...], preferred_element_type=jnp.float32) + b0_ref[...]
    x0_ref[...] = x0.astype(jnp.bfloat16)
    f1 = _l2n_rows(f1_ref[...])
    x1 = jnp.dot(f1, W1_ref[...], preferred_element_type=jnp.float32) + b1_ref[...]
    x1_ref[...] = x1.astype(jnp.bfloat16)
    st = jnp.concatenate(
        [jnp.sum(x0, axis=0, keepdims=True),
         jnp.sum(x0 * x0, axis=0, keepdims=True),
         jnp.sum(x1, axis=0, keepdims=True),
         jnp.sum(x1 * x1, axis=0, keepdims=True)], axis=0)

    @pl.when(i == 0)
    def _():
        st_ref[...] = st

    @pl.when(i > 0)
    def _():
        st_ref[...] = st_ref[...] + st


def _p2_body(x0_ref, x1_ref, st_ref, pv_ref, Wf_ref, bf_ref, nf_ref, nfh_ref):
    st = st_ref[...]
    inv_n = 1.0 / N_ITEMS
    mu0 = st[0:1, :] * inv_n
    var0 = st[1:2, :] * inv_n - mu0 * mu0
    mu1 = st[2:3, :] * inv_n
    var1 = st[3:4, :] * inv_n - mu1 * mu1
    p = pv_ref[...]
    g0, be0 = p[0:1, :], p[1:2, :]
    g1, be1 = p[2:3, :], p[3:4, :]
    fa, fb = p[4:5, :], p[5:6, :]
    m = jnp.maximum(fa, fb)
    ea = jnp.exp(fa - m)
    eb = jnp.exp(fb - m)
    w0 = ea / (ea + eb)
    w1 = eb / (ea + eb)
    x0 = x0_ref[...].astype(jnp.float32)
    x1 = x1_ref[...].astype(jnp.float32)
    h0 = jax.nn.relu(g0 * (x0 - mu0) / jnp.sqrt(var0 + _EPS_BN) + be0)
    h1 = jax.nn.relu(g1 * (x1 - mu1) / jnp.sqrt(var1 + _EPS_BN) + be1)
    fused = w0 * h0 + w1 * h1
    fo = jnp.dot(fused, Wf_ref[...], preferred_element_type=jnp.float32) + bf_ref[...]
    nf = _l2n_rows(fo)
    nf_ref[...] = nf
    nfh_ref[0] = nf[:, 0:HEMB]
    nfh_ref[1] = nf[:, HEMB:EMB]


def _p3_body(pa_ref, out_ref):
    u = jnp.concatenate([pa_ref[0], pa_ref[1]], axis=-1)
    out_ref[...] = _l2n_rows(u)


def _sc_body(nfh_hbm, warm_hbm, rows_hbm, cols_hbm, vals_hbm, zeros_hbm,
             out_hbm,
             rows_v, cols_v, widx_v, vals_v, gath_v, zb_v, wtmp_v,
             warm_sp, acc, wsem, nfsem, scsem):
    cid = lax.axis_index("c")
    sid = lax.axis_index("s")

    @pl.when(sid < 8)
    def _():
        pltpu.sync_copy(warm_hbm.at[pl.ds(sid * 5000, 5000)], wtmp_v)
        pltpu.sync_copy(wtmp_v, warm_sp.at[pl.ds(sid * 5000, 5000)])
    pltpu.sync_copy(zeros_hbm, zb_v)
    for q in range(NWCH):
        pltpu.sync_copy(zb_v, acc.at[pl.ds(sid * RPT + q * WCH, WCH)])
    plsc.subcore_barrier()

    def prep_a(kk, bb):
        rowbase = sid * ROWS_PT + kk * NSUB
        pltpu.sync_copy(rows_hbm.at[pl.ds(rowbase, NSUB)], rows_v.at[bb])
        pltpu.sync_copy(vals_hbm.at[pl.ds(rowbase, NSUB)], vals_v.at[bb])
        pltpu.sync_copy(cols_hbm.at[pl.ds(rowbase, NSUB)], cols_v.at[bb])
        for j in range(NSUB):
            pltpu.async_copy(warm_sp.at[cols_v.at[bb, j]],
                             widx_v.at[bb, j], wsem)

    def prep_b(bb):
        for j in range(NSUB):
            pltpu.make_async_copy(warm_sp.at[cols_v.at[bb, j]],
                                  widx_v.at[bb, j], wsem).wait()
        for j in range(NSUB):
            pltpu.async_copy(nfh_hbm.at[cid].at[widx_v.at[bb, j]],
                             gath_v.at[bb, j], nfsem)

    def drain_nf(bb):
        for j in range(NSUB):
            pltpu.make_async_copy(nfh_hbm.at[cid].at[widx_v.at[bb, j]],
                                  gath_v.at[bb, j], nfsem).wait()

    def scale(bb):
        for j in range(NSUB):
            def sgroup(g, _):
                vv = vals_v[bb, j, pl.ds(g * 16, 16)]
                for t in range(16):
                    e = g * 16 + t
                    v = vv[t]
                    gath_v[bb, j, e, 0:HEMB] = gath_v[bb, j, e, 0:HEMB] * v
                return 0

            lax.fori_loop(0, SUBW // 16, sgroup, 0)

    def fire_scat(bb):
        for j in range(NSUB):
            pltpu.async_copy(gath_v.at[bb, j], acc.at[rows_v.at[bb, j]],
                             scsem, add=True)

    def drain_scat(bb):
        for j in range(NSUB):
            pltpu.make_async_copy(gath_v.at[bb, j], acc.at[rows_v.at[bb, j]],
                                  scsem).wait()

    prep_a(0, 0)
    prep_a(1, 1)
    prep_b(0)
    drain_nf(0)
    scale(0)
    fire_scat(0)
    prep_b(1)
    prep_a(2, 2)

    def step(k, _):
        b = k % NBUF
        drain_nf(b)
        scale(b)
        fire_scat(b)
        drain_scat(b)
        prep_b((k + 1) % NBUF)
        prep_a(k + 2, (k + 2) % NBUF)
        return 0

    lax.fori_loop(1, NCHUNK - 2, step, 0)
    b = (NCHUNK - 2) % NBUF
    drain_nf(b)
    scale(b)
    fire_scat(b)
    drain_scat(b)
    prep_b((NCHUNK - 1) % NBUF)
    b = (NCHUNK - 1) % NBUF
    drain_nf(b)
    scale(b)
    fire_scat(b)
    for q in range(2):
        drain_scat(q)

    plsc.subcore_barrier()
    for q in range(NWCH):
        base = sid * RPT + q * WCH
        pltpu.sync_copy(acc.at[pl.ds(base, WCH)], zb_v)
        pltpu.sync_copy(zb_v, out_hbm.at[cid, pl.ds(base, WCH)])


def kernel(feats0, feats1, adj_rows, adj_cols, adj_vals, warm_idx,
           W0, b0, g0, be0, W1, b1, g1, be1, fuse_w, Wf, bf):
    f32 = jnp.float32

    x0, x1, st = pl.pallas_call(
        _p1_body,
        grid=(NB,),
        in_specs=[
            pl.BlockSpec((R, D0), lambda i: (i, 0)),
            pl.BlockSpec((R, D1), lambda i: (i, 0)),
            pl.BlockSpec((D0, HID), lambda i: (0, 0)),
            pl.BlockSpec((1, HID), lambda i: (0, 0)),
            pl.BlockSpec((D1, HID), lambda i: (0, 0)),
            pl.BlockSpec((1, HID), lambda i: (0, 0)),
        ],
        out_specs=[
            pl.BlockSpec((R, HID), lambda i: (i, 0)),
            pl.BlockSpec((R, HID), lambda i: (i, 0)),
            pl.BlockSpec((4, HID), lambda i: (0, 0)),
        ],
        out_shape=[
            jax.ShapeDtypeStruct((N_ITEMS, HID), jnp.bfloat16),
            jax.ShapeDtypeStruct((N_ITEMS, HID), jnp.bfloat16),
            jax.ShapeDtypeStruct((4, HID), f32),
        ],
    )(feats0, feats1, W0, b0.reshape(1, HID), W1, b1.reshape(1, HID))

    pv = jnp.stack([g0, be0, g1, be1,
                    jnp.full((HID,), fuse_w[0], dtype=f32),
                    jnp.full((HID,), fuse_w[1], dtype=f32)], axis=0)

    nf, nfh = pl.pallas_call(
        _p2_body,
        grid=(NB,),
        in_specs=[
            pl.BlockSpec((R, HID), lambda i: (i, 0)),
            pl.BlockSpec((R, HID), lambda i: (i, 0)),
            pl.BlockSpec((4, HID), lambda i: (0, 0)),
            pl.BlockSpec((6, HID), lambda i: (0, 0)),
            pl.BlockSpec((HID, EMB), lambda i: (0, 0)),
            pl.BlockSpec((1, EMB), lambda i: (0, 0)),
        ],
        out_specs=[
            pl.BlockSpec((R, EMB), lambda i: (i, 0)),
            pl.BlockSpec((NC, R, HEMB), lambda i: (0, i, 0)),
        ],
        out_shape=[
            jax.ShapeDtypeStruct((N_ITEMS, EMB), f32),
            jax.ShapeDtypeStruct((NC, N_ITEMS, HEMB), f32),
        ],
    )(x0, x1, st, pv, Wf, bf.reshape(1, EMB))

    pad = N_EDGES_P - N_EDGES
    i32 = jnp.int32
    rows2 = jnp.concatenate(
        [adj_rows.astype(i32), jnp.zeros((pad,), i32)]).reshape(EROWS, SUBW)
    cols2 = jnp.concatenate(
        [adj_cols.astype(i32), jnp.zeros((pad,), i32)]).reshape(EROWS, SUBW)
    vals2 = jnp.concatenate(
        [adj_vals, jnp.zeros((pad,), f32)]).reshape(EROWS, SUBW)
    zeros = jnp.zeros((WCH, HEMB), dtype=f32)

    mesh = plsc.VectorSubcoreMesh(core_axis_name="c", subcore_axis_name="s")
    partials = pl.kernel(
        _sc_body,
        mesh=mesh,
        compiler_params=pltpu.CompilerParams(use_tc_tiling_on_sc=False, needs_layout_passes=False),
        out_type=jax.ShapeDtypeStruct((NC, N_USERS_P, HEMB), f32),
        scratch_types=[
            pltpu.VMEM((NBUF, NSUB, SUBW), jnp.int32),
            pltpu.VMEM((NBUF, NSUB, SUBW), jnp.int32),
            pltpu.VMEM((NBUF, NSUB, SUBW), jnp.int32),
            pltpu.VMEM((NBUF, NSUB, SUBW), f32),
            pltpu.VMEM((NBUF, NSUB, SUBW, HEMB), f32),
            pltpu.VMEM((WCH, HEMB), f32),
            pltpu.VMEM((5000,), jnp.int32),
            pltpu.VMEM_SHARED((N_WARM,), jnp.int32),
            pltpu.VMEM_SHARED((N_USERS_P, HEMB), f32),
            pltpu.SemaphoreType.DMA,
            pltpu.SemaphoreType.DMA,
            pltpu.SemaphoreType.DMA,
        ],
    )(nfh, warm_idx.astype(jnp.int32), rows2, cols2, vals2, zeros)

    user_vecs = pl.pallas_call(
        _p3_body,
        grid=(NB,),
        in_specs=[pl.BlockSpec((NC, R, HEMB), lambda i: (0, i, 0))],
        out_specs=pl.BlockSpec((R, EMB), lambda i: (i, 0)),
        out_shape=jax.ShapeDtypeStruct((N_USERS, EMB), f32),
    )(partials)

    return (user_vecs, nf)


# speedup vs baseline: 1.1813x; 1.1813x over previous
import functools

import jax
import jax.numpy as jnp
from jax import lax
from jax.experimental import pallas as pl
from jax.experimental.pallas import tpu as pltpu
from jax.experimental.pallas import tpu_sc as plsc

N_USERS = 50000
N_ITEMS = 50000
N_WARM = 40000
N_EDGES = 1600000
D0 = 256
D1 = 128
HID = 128
EMB = 32

R = 2000
NB = N_ITEMS // R

NC = 2
NS = 16
NW = NC * NS
SUBW = 128
NSUB = 10
C = SUBW * NSUB
NCHUNK = 80
EPT = C * NCHUNK
N_EDGES_P = EPT * NS
ROWS_PT = EPT // SUBW
HEMB = EMB // NC
NBUF = 3
WNS = 16
WCHUNK = (N_EDGES_P // SUBW) // NW // WNS
RPT = 3128
N_USERS_P = RPT * NS
WCH = 184
NWCH = RPT // WCH
EROWS = N_EDGES_P // SUBW

_EPS_N = 1e-12
_EPS_BN = 1e-5


def _l2n_rows(x):
    n = jnp.sqrt(jnp.sum(x * x, axis=-1, keepdims=True))
    return x / jnp.maximum(n, _EPS_N)


def _p1_body(f0_ref, f1_ref, W0_ref, b0_ref, W1_ref, b1_ref,
             x0_ref, x1_ref, st_ref):
    i = pl.program_id(0)
    f0 = _l2n_rows(f0_ref[...])
    x0 = jnp.dot(f0, W0_ref[...], preferred_element_type=jnp.float32) + b0_ref[...]
    x0_ref[...] = x0.astype(jnp.bfloat16)
    f1 = _l2n_rows(f1_ref[...])
    x1 = jnp.dot(f1, W1_ref[...], preferred_element_type=jnp.float32) + b1_ref[...]
    x1_ref[...] = x1.astype(jnp.bfloat16)
    st = jnp.concatenate(
        [jnp.sum(x0, axis=0, keepdims=True),
         jnp.sum(x0 * x0, axis=0, keepdims=True),
         jnp.sum(x1, axis=0, keepdims=True),
         jnp.sum(x1 * x1, axis=0, keepdims=True)], axis=0)

    @pl.when(i == 0)
    def _():
        st_ref[...] = st

    @pl.when(i > 0)
    def _():
        st_ref[...] = st_ref[...] + st


def _p2_body(x0_ref, x1_ref, st_ref, pv_ref, Wf_ref, bf_ref, nf_ref, nfh_ref):
    st = st_ref[...]
    inv_n = 1.0 / N_ITEMS
    mu0 = st[0:1, :] * inv_n
    var0 = st[1:2, :] * inv_n - mu0 * mu0
    mu1 = st[2:3, :] * inv_n
    var1 = st[3:4, :] * inv_n - mu1 * mu1
    p = pv_ref[...]
    g0, be0 = p[0:1, :], p[1:2, :]
    g1, be1 = p[2:3, :], p[3:4, :]
    fa, fb = p[4:5, :], p[5:6, :]
    m = jnp.maximum(fa, fb)
    ea = jnp.exp(fa - m)
    eb = jnp.exp(fb - m)
    w0 = ea / (ea + eb)
    w1 = eb / (ea + eb)
    x0 = x0_ref[...].astype(jnp.float32)
    x1 = x1_ref[...].astype(jnp.float32)
    h0 = jax.nn.relu(g0 * (x0 - mu0) / jnp.sqrt(var0 + _EPS_BN) + be0)
    h1 = jax.nn.relu(g1 * (x1 - mu1) / jnp.sqrt(var1 + _EPS_BN) + be1)
    fused = w0 * h0 + w1 * h1
    fo = jnp.dot(fused, Wf_ref[...], preferred_element_type=jnp.float32) + bf_ref[...]
    nf = _l2n_rows(fo)
    nf_ref[...] = nf
    nfh_ref[0] = nf[:, 0:HEMB]
    nfh_ref[1] = nf[:, HEMB:EMB]


def _p3_body(pa_ref, out_ref):
    u = jnp.concatenate([pa_ref[0], pa_ref[1]], axis=-1)
    out_ref[...] = _l2n_rows(u)


def _widx_body(warm_hbm, cols_hbm, widx_hbm, cbuf, wbuf, warm_v, sem):
    cid = lax.axis_index("c")
    sid = lax.axis_index("s")
    wid = sid * NC + cid

    pltpu.sync_copy(warm_hbm, warm_v)

    def wchunk(k, _):
        rowbase = wid * (WNS * WCHUNK) + k * WNS
        pltpu.sync_copy(cols_hbm.at[pl.ds(rowbase, WNS)], cbuf)
        for j in range(WNS):
            for g in range(SUBW // 16):
                idx16 = cbuf[j, pl.ds(g * 16, 16)]
                wbuf[j, pl.ds(g * 16, 16)] = plsc.load_gather(warm_v, [idx16])
        pltpu.sync_copy(wbuf, widx_hbm.at[pl.ds(rowbase, WNS)])
        return 0

    lax.fori_loop(0, WCHUNK, wchunk, 0)


def _sc_body(nfh_hbm, rows_hbm, widx_hbm, vals_hbm, zeros_hbm,
             out_hbm,
             rows_v, widx_v, vals_v, gath_v, zb_v, acc, nfsem, scsem):
    cid = lax.axis_index("c")
    sid = lax.axis_index("s")

    pltpu.sync_copy(zeros_hbm, zb_v)
    for q in range(NWCH):
        pltpu.sync_copy(zb_v, acc.at[pl.ds(sid * RPT + q * WCH, WCH)])
    plsc.subcore_barrier()

    def prep(kk, bb):
        rowbase = sid * ROWS_PT + kk * NSUB
        pltpu.sync_copy(rows_hbm.at[pl.ds(rowbase, NSUB)], rows_v.at[bb])
        pltpu.sync_copy(vals_hbm.at[pl.ds(rowbase, NSUB)], vals_v.at[bb])
        pltpu.sync_copy(widx_hbm.at[pl.ds(rowbase, NSUB)], widx_v.at[bb])
        for j in range(NSUB):
            pltpu.async_copy(nfh_hbm.at[cid].at[widx_v.at[bb, j]],
                             gath_v.at[bb, j], nfsem)

    def drain_nf(bb):
        for j in range(NSUB):
            pltpu.make_async_copy(nfh_hbm.at[cid].at[widx_v.at[bb, j]],
                                  gath_v.at[bb, j], nfsem).wait()

    def scale(bb):
        for j in range(NSUB):
            def sgroup(g, _):
                vv = vals_v[bb, j, pl.ds(g * 16, 16)]
                for t in range(16):
                    e = g * 16 + t
                    v = vv[t]
                    gath_v[bb, j, e, 0:HEMB] = gath_v[bb, j, e, 0:HEMB] * v
                return 0

            lax.fori_loop(0, SUBW // 16, sgroup, 0)

    def fire_scat(bb):
        for j in range(NSUB):
            pltpu.async_copy(gath_v.at[bb, j], acc.at[rows_v.at[bb, j]],
                             scsem, add=True)

    def drain_scat(bb):
        for j in range(NSUB):
            pltpu.make_async_copy(gath_v.at[bb, j], acc.at[rows_v.at[bb, j]],
                                  scsem).wait()

    prep(0, 0)
    prep(1, 1)
    drain_nf(0)
    scale(0)
    fire_scat(0)
    prep(2, 2)

    def step(k, _):
        b = k % NBUF
        bb = (k + 2) % NBUF
        drain_nf(b)
        scale(b)
        fire_scat(b)
        drain_scat(b)
        prep(k + 2, bb)
        return 0

    lax.fori_loop(1, NCHUNK - 2, step, 0)
    for k in (NCHUNK - 2, NCHUNK - 1):
        b = k % NBUF
        drain_nf(b)
        scale(b)
        fire_scat(b)
    for q in range(NBUF):
        drain_scat(q)

    plsc.subcore_barrier()
    for q in range(NWCH):
        base = sid * RPT + q * WCH
        pltpu.sync_copy(acc.at[pl.ds(base, WCH)], zb_v)
        pltpu.sync_copy(zb_v, out_hbm.at[cid, pl.ds(base, WCH)])


def kernel(feats0, feats1, adj_rows, adj_cols, adj_vals, warm_idx,
           W0, b0, g0, be0, W1, b1, g1, be1, fuse_w, Wf, bf):
    f32 = jnp.float32

    x0, x1, st = pl.pallas_call(
        _p1_body,
        grid=(NB,),
        in_specs=[
            pl.BlockSpec((R, D0), lambda i: (i, 0)),
            pl.BlockSpec((R, D1), lambda i: (i, 0)),
            pl.BlockSpec((D0, HID), lambda i: (0, 0)),
            pl.BlockSpec((1, HID), lambda i: (0, 0)),
            pl.BlockSpec((D1, HID), lambda i: (0, 0)),
            pl.BlockSpec((1, HID), lambda i: (0, 0)),
        ],
        out_specs=[
            pl.BlockSpec((R, HID), lambda i: (i, 0)),
            pl.BlockSpec((R, HID), lambda i: (i, 0)),
            pl.BlockSpec((4, HID), lambda i: (0, 0)),
        ],
        out_shape=[
            jax.ShapeDtypeStruct((N_ITEMS, HID), jnp.bfloat16),
            jax.ShapeDtypeStruct((N_ITEMS, HID), jnp.bfloat16),
            jax.ShapeDtypeStruct((4, HID), f32),
        ],
    )(feats0, feats1, W0, b0.reshape(1, HID), W1, b1.reshape(1, HID))

    pv = jnp.stack([g0, be0, g1, be1,
                    jnp.full((HID,), fuse_w[0], dtype=f32),
                    jnp.full((HID,), fuse_w[1], dtype=f32)], axis=0)

    nf, nfh = pl.pallas_call(
        _p2_body,
        grid=(NB,),
        in_specs=[
            pl.BlockSpec((R, HID), lambda i: (i, 0)),
            pl.BlockSpec((R, HID), lambda i: (i, 0)),
            pl.BlockSpec((4, HID), lambda i: (0, 0)),
            pl.BlockSpec((6, HID), lambda i: (0, 0)),
            pl.BlockSpec((HID, EMB), lambda i: (0, 0)),
            pl.BlockSpec((1, EMB), lambda i: (0, 0)),
        ],
        out_specs=[
            pl.BlockSpec((R, EMB), lambda i: (i, 0)),
            pl.BlockSpec((NC, R, HEMB), lambda i: (0, i, 0)),
        ],
        out_shape=[
            jax.ShapeDtypeStruct((N_ITEMS, EMB), f32),
            jax.ShapeDtypeStruct((NC, N_ITEMS, HEMB), f32),
        ],
    )(x0, x1, st, pv, Wf, bf.reshape(1, EMB))

    pad = N_EDGES_P - N_EDGES
    i32 = jnp.int32
    rows2 = jnp.concatenate(
        [adj_rows.astype(i32), jnp.zeros((pad,), i32)]).reshape(EROWS, SUBW)
    cols2 = jnp.concatenate(
        [adj_cols.astype(i32), jnp.zeros((pad,), i32)]).reshape(EROWS, SUBW)
    vals2 = jnp.concatenate(
        [adj_vals, jnp.zeros((pad,), f32)]).reshape(EROWS, SUBW)
    zeros = jnp.zeros((WCH, HEMB), dtype=f32)

    mesh = plsc.VectorSubcoreMesh(core_axis_name="c", subcore_axis_name="s")
    widx2 = pl.kernel(
        _widx_body,
        mesh=mesh,
        compiler_params=pltpu.CompilerParams(use_tc_tiling_on_sc=False, needs_layout_passes=False),
        out_type=jax.ShapeDtypeStruct((EROWS, SUBW), jnp.int32),
        scratch_types=[
            pltpu.VMEM((WNS, SUBW), jnp.int32),
            pltpu.VMEM((WNS, SUBW), jnp.int32),
            pltpu.VMEM((N_WARM,), jnp.int32),
            pltpu.SemaphoreType.DMA,
        ],
    )(warm_idx.astype(jnp.int32), cols2)

    partials = pl.kernel(
        _sc_body,
        mesh=mesh,
        compiler_params=pltpu.CompilerParams(use_tc_tiling_on_sc=False, needs_layout_passes=False),
        out_type=jax.ShapeDtypeStruct((NC, N_USERS_P, HEMB), f32),
        scratch_types=[
            pltpu.VMEM((NBUF, NSUB, SUBW), jnp.int32),
            pltpu.VMEM((NBUF, NSUB, SUBW), jnp.int32),
            pltpu.VMEM((NBUF, NSUB, SUBW), f32),
            pltpu.VMEM((NBUF, NSUB, SUBW, HEMB), f32),
            pltpu.VMEM((WCH, HEMB), f32),
            pltpu.VMEM_SHARED((N_USERS_P, HEMB), f32),
            pltpu.SemaphoreType.DMA,
            pltpu.SemaphoreType.DMA,
        ],
    )(nfh, rows2, widx2, vals2, zeros)

    user_vecs = pl.pallas_call(
        _p3_body,
        grid=(NB,),
        in_specs=[pl.BlockSpec((NC, R, HEMB), lambda i: (0, i, 0))],
        out_specs=pl.BlockSpec((R, EMB), lambda i: (i, 0)),
        out_shape=jax.ShapeDtypeStruct((N_USERS, EMB), f32),
    )(partials)

    return (user_vecs, nf)
